# baseline (speedup 1.0000x reference)
import functools

import jax
import jax.numpy as jnp
from jax import lax
from jax.experimental import pallas as pl
from jax.experimental.pallas import tpu as pltpu, tpu_sc as plsc

_info = plsc.get_sparse_core_info()
_NC = _info.num_cores
_NS = _info.num_subcores
_NW = _NC * _NS

_CHUNK = 256


@functools.lru_cache(maxsize=None)
def _make_gather(B: int, D: int):
    assert D == 64
    assert B % (2 * _CHUNK * _NW) == 0
    b_per_w = B // _NW
    n_chunks = b_per_w // _CHUNK
    ochunk = _CHUNK // 2

    mesh = plsc.VectorSubcoreMesh(core_axis_name="c", subcore_axis_name="s")

    @functools.partial(
        pl.kernel,
        out_type=jax.ShapeDtypeStruct((B // 2, 2 * D), jnp.float32),
        mesh=mesh,
        scratch_types=(
            [pltpu.VMEM((_CHUNK,), jnp.int32) for _ in range(2)]
            + [pltpu.VMEM((_CHUNK,), jnp.int32) for _ in range(2)]
            + [pltpu.VMEM((_CHUNK, 2 * D), jnp.float32) for _ in range(2)]
            + [pltpu.VMEM((ochunk, 2 * D), jnp.float32) for _ in range(2)]
            + [pltpu.SemaphoreType.DMA for _ in range(4)]
        ),
        compiler_params=pltpu.CompilerParams(use_tc_tiling_on_sc=True),
    )
    def gather_kernel(table2_hbm, ih_hbm, io_hbm, out_hbm, *refs):
        ih = refs[0:2]
        io = refs[2:4]
        rows = refs[4:6]
        obuf = refs[6:8]
        gsem = refs[8:10]
        osem = refs[10:12]

        wid = lax.axis_index("s") * _NC + lax.axis_index("c")
        base = wid * b_per_w
        obase = base // 2

        def stage_idx(c, b):
            off = pl.multiple_of(base + c * _CHUNK, _CHUNK)
            pltpu.sync_copy(ih_hbm.at[pl.ds(off, _CHUNK)], ih[b])
            pltpu.sync_copy(io_hbm.at[pl.ds(off, _CHUNK)], io[b])

        def gather_copy(b):
            return pltpu.make_async_copy(table2_hbm.at[ih[b]], rows[b], gsem[b])

        def out_copy(c, b):
            return pltpu.make_async_copy(
                obuf[b],
                out_hbm.at[pl.ds(pl.multiple_of(obase + c * ochunk, ochunk), ochunk)],
                osem[b],
            )

        stage_idx(0, 0)
        gather_copy(0).start()

        def step(c, b):
            @pl.when(c + 1 < n_chunks)
            def _():
                stage_idx(c + 1, 1 - b)
                gather_copy(1 - b).start()

            gather_copy(b).wait()

            @pl.when(c >= 2)
            def _():
                out_copy(c - 2, b).wait()

            def group16(g, carry):
                offv = io[b][pl.ds(pl.multiple_of(16 * g, 16), 16)]
                rg = rows[b].at[pl.ds(pl.multiple_of(16 * g, 16), 16)]
                og = obuf[b].at[pl.ds(pl.multiple_of(8 * g, 8), 8)]
                for l in range(16):
                    o = offv[l]
                    for k in range(4):
                        og[l // 2, pl.ds((l % 2) * 64 + 16 * k, 16)] = rg[
                            l, pl.ds(o + 16 * k, 16)
                        ]
                return carry

            lax.fori_loop(0, _CHUNK // 16, group16, 0)
            out_copy(c, b).start()

        def group(g, carry):
            step(2 * g, 0)
            step(2 * g + 1, 1)
            return carry

        lax.fori_loop(0, n_chunks // 2, group, 0)

        out_copy(n_chunks - 2, 0).wait()
        out_copy(n_chunks - 1, 1).wait()

    return gather_kernel


def kernel(indices, table):
    B = indices.size
    V, D = table.shape
    flat = indices.reshape(B).astype(jnp.int32)
    table2 = table.reshape(V // 2, 2 * D)
    idx_half = flat >> 1
    idx_off = (flat & 1) << 6
    out2 = _make_gather(B, D)(table2, idx_half, idx_off)
    return out2.reshape(indices.shape + (D,))


# speedup vs baseline: 1.3379x; 1.3379x over previous
import functools

import jax
import jax.numpy as jnp
from jax import lax
from jax.experimental import pallas as pl
from jax.experimental.pallas import tpu as pltpu, tpu_sc as plsc

_info = plsc.get_sparse_core_info()
_NC = _info.num_cores
_NS = _info.num_subcores
_NW = _NC * _NS

_NBUF = 4
_LOOKAHEAD = 2


@functools.lru_cache(maxsize=None)
def _make_gather(B: int, D: int):
    assert B % (8 * _NW) == 0
    b_per_w = B // _NW
    chunk = 320
    while b_per_w % (chunk * _NBUF):
        chunk //= 2
    n_chunks = b_per_w // chunk
    n_groups = n_chunks // _NBUF

    mesh = plsc.VectorSubcoreMesh(core_axis_name="c", subcore_axis_name="s")

    @functools.partial(
        pl.kernel,
        out_type=jax.ShapeDtypeStruct((B, D), jnp.float32),
        mesh=mesh,
        scratch_types=[
            pltpu.VMEM((b_per_w,), jnp.int32),
        ]
        + [pltpu.VMEM((chunk, D), jnp.float32) for _ in range(_NBUF)]
        + [pltpu.SemaphoreType.DMA for _ in range(2 * _NBUF)],
        compiler_params=pltpu.CompilerParams(use_tc_tiling_on_sc=False),
    )
    def gather_kernel(table_hbm, idx_hbm, out_hbm, idx_v, *bufs_and_sems):
        rows = bufs_and_sems[:_NBUF]
        gsem = bufs_and_sems[_NBUF : 2 * _NBUF]
        ssem = bufs_and_sems[2 * _NBUF :]

        wid = lax.axis_index("s") * _NC + lax.axis_index("c")
        base = wid * b_per_w
        pltpu.sync_copy(idx_hbm.at[pl.ds(base, b_per_w)], idx_v)

        def gather_copy(c, b):
            return pltpu.make_async_copy(
                table_hbm.at[idx_v.at[pl.ds(c * chunk, chunk)]], rows[b], gsem[b]
            )

        def scatter_copy(c, b):
            return pltpu.make_async_copy(
                rows[b], out_hbm.at[pl.ds(base + c * chunk, chunk)], ssem[b]
            )

        for b in range(_LOOKAHEAD):
            gather_copy(b, b).start()

        def group(g, carry):
            for b in range(_NBUF):
                c = g * _NBUF + b
                gather_copy(c, b).wait()
                scatter_copy(c, b).start()
                b2 = (b + _LOOKAHEAD) % _NBUF
                c2 = c + _LOOKAHEAD

                @pl.when(c2 < n_chunks)
                def _():
                    @pl.when(c2 >= _NBUF)
                    def _():
                        scatter_copy(c2 - _NBUF, b2).wait()

                    gather_copy(c2, b2).start()

            return carry

        lax.fori_loop(0, n_groups, group, 0)

        for b in range(_NBUF):
            scatter_copy(n_chunks - _NBUF + b, b).wait()

    return gather_kernel


def kernel(indices, table):
    B = indices.size
    nb, ns = indices.shape
    flat = indices.T.reshape(B).astype(jnp.int32)
    out = _make_gather(B, table.shape[1])(table, flat)
    return out.reshape(ns, nb, table.shape[1]).transpose(1, 0, 2)


# speedup vs baseline: 1.3503x; 1.0093x over previous
import functools

import jax
import jax.numpy as jnp
from jax import lax
from jax.experimental import pallas as pl
from jax.experimental.pallas import tpu as pltpu, tpu_sc as plsc

_info = plsc.get_sparse_core_info()
_NC = _info.num_cores
_NS = _info.num_subcores
_NW = _NC * _NS

_CHUNK = 256


@functools.lru_cache(maxsize=None)
def _make_gather(ns: int, nb: int, D: int):
    assert D == 64
    b_per_w = nb // _NW
    hpw = b_per_w // _CHUNK
    n_chunks = ns * hpw

    mesh = plsc.VectorSubcoreMesh(core_axis_name="c", subcore_axis_name="s")

    @functools.partial(
        pl.kernel,
        out_type=jax.ShapeDtypeStruct((ns, D, nb), jnp.float32),
        mesh=mesh,
        scratch_types=(
            [pltpu.VMEM((_CHUNK,), jnp.int32) for _ in range(2)]
            + [pltpu.VMEM((_CHUNK,), jnp.int32) for _ in range(2)]
            + [pltpu.VMEM((_CHUNK, 2 * D), jnp.float32) for _ in range(2)]
            + [pltpu.VMEM((D, _CHUNK), jnp.float32) for _ in range(2)]
            + [pltpu.SemaphoreType.DMA for _ in range(4)]
        ),
        compiler_params=pltpu.CompilerParams(
            use_tc_tiling_on_sc=True, needs_layout_passes=False
        ),
    )
    def gather_kernel(table2_hbm, ih_hbm, io_hbm, out_hbm, *refs):
        ih = refs[0:2]
        io = refs[2:4]
        rows = refs[4:6]
        tb = refs[6:8]
        gsem = refs[8:10]
        osem = refs[10:12]

        wid = lax.axis_index("s") * _NC + lax.axis_index("c")
        bbase = wid * b_per_w

        iota = lax.iota(jnp.int32, 16)
        tj = [(iota + j) & 15 for j in range(16)]

        def split(c):
            s = c // hpw
            b0 = bbase + (c % hpw) * _CHUNK
            return s, b0

        def stage_idx(c, b):
            s, b0 = split(c)
            p0 = pl.multiple_of(s * nb + b0, _CHUNK)
            pltpu.sync_copy(ih_hbm.at[pl.ds(p0, _CHUNK)], ih[b])
            pltpu.sync_copy(io_hbm.at[pl.ds(p0, _CHUNK)], io[b])

        def gather_copy(b):
            return pltpu.make_async_copy(table2_hbm.at[ih[b]], rows[b], gsem[b])

        def out_copy(c, b):
            s, b0 = split(c)
            return pltpu.make_async_copy(
                tb[b],
                out_hbm.at[s, :, pl.ds(pl.multiple_of(b0, _CHUNK), _CHUNK)],
                osem[b],
            )

        stage_idx(0, 0)
        gather_copy(0).start()

        def step(c, b):
            @pl.when(c + 1 < n_chunks)
            def _():
                stage_idx(c + 1, 1 - b)
                gather_copy(1 - b).start()

            gather_copy(b).wait()

            @pl.when(c >= 2)
            def _():
                out_copy(c - 2, b).wait()

            def blk(R, carry):
                rr = R * 16 + iota
                iov = io[b][pl.ds(pl.multiple_of(R * 16, 16), 16)]
                for j in range(16):
                    sc = iov + tj[j]
                    for C in range(4):
                        vals = plsc.load_gather(rows[b], [rr, sc + (16 * C)])
                        plsc.store_scatter(tb[b], [tj[j] + (16 * C), rr], vals)
                return carry

            lax.fori_loop(0, _CHUNK // 16, blk, 0)
            out_copy(c, b).start()

        def pair(g, carry):
            step(2 * g, 0)
            step(2 * g + 1, 1)
            return carry

        lax.fori_loop(0, n_chunks // 2, pair, 0)

        out_copy(n_chunks - 2, 0).wait()
        out_copy(n_chunks - 1, 1).wait()

    return gather_kernel


def kernel(indices, table):
    nb, ns = indices.shape
    V, D = table.shape
    flat = indices.T.reshape(nb * ns).astype(jnp.int32)
    table2 = table.reshape(V // 2, 2 * D)
    ih = flat >> 1
    io = (flat & 1) << 6
    out = _make_gather(ns, nb, D)(table2, ih, io)
    return out.transpose(2, 0, 1)
